# baseline (speedup 1.0000x reference)
import functools

import jax
import jax.numpy as jnp
from jax import lax
from jax.experimental import pallas as pl
from jax.experimental.pallas import tpu as pltpu
from jax.experimental.pallas import tpu_sc as plsc

N_NODES = 10000
N_EDGES = 320000
DIM = 128
HALF = DIM // 2

NC = 2
NS = 16
CHUNK = 128
CHUNKS_PER_TILE = 80
EDGES_PER_TILE = CHUNK * CHUNKS_PER_TILE
E_PAD = NC * NS * EDGES_PER_TILE
ACC_ROWS = 10112
ROWS_PER_TILE = ACC_ROWS // NS

NSTREAM = 4
CPS = CHUNKS_PER_TILE // NSTREAM


def _sc_aggregate(x2, idx_t, zeros_init):
  mesh = plsc.VectorSubcoreMesh(
      core_axis_name="c", subcore_axis_name="s", num_cores=NC,
      num_subcores=NS)

  @functools.partial(
      pl.kernel,
      out_type=jax.ShapeDtypeStruct((NC, 2, ACC_ROWS, HALF), jnp.float32),
      mesh=mesh,
      scratch_types=[
          pltpu.VMEM_SHARED((ACC_ROWS, HALF), jnp.float32),
          pltpu.VMEM_SHARED((ACC_ROWS, HALF), jnp.float32),
          pltpu.VMEM((NSTREAM * 2, 2, CHUNK), jnp.int32),
          pltpu.VMEM((NSTREAM, CHUNK, HALF), jnp.float32),
          [pltpu.SemaphoreType.DMA] * NSTREAM,
          [pltpu.SemaphoreType.DMA] * NSTREAM,
          [pltpu.SemaphoreType.DMA] * NSTREAM,
      ],
      compiler_params=pltpu.CompilerParams(use_tc_tiling_on_sc=False),
  )
  def sc_kernel(x_hbm, idx_hbm, zer_hbm, out_hbm,
                x_sp, acc, idxb, rows, gsems, ssems, isems):
    c = lax.axis_index("c")
    s = lax.axis_index("s")
    rslice = pl.ds(s * ROWS_PER_TILE, ROWS_PER_TILE)

    def fire_idx(t, i, pb):
      pltpu.async_copy(idx_hbm.at[c, s, t * CPS + i],
                       idxb.at[t * 2 + pb], isems[t])

    def wait_idx(t, pb):
      pltpu.make_async_copy(idx_hbm.at[0, 0, 0], idxb.at[t * 2 + pb],
                            isems[t]).wait()

    def fire_gather(t, pb, p):
      pltpu.async_copy(x_sp.at[idxb.at[t * 2 + pb, 0]],
                       rows.at[t], gsems[t])

    def wait_gather(t, pb, p):
      pltpu.make_async_copy(x_sp.at[idxb.at[t * 2 + pb, 0]],
                            rows.at[t], gsems[t]).wait()

    def fire_scatter(t, pb):
      pltpu.async_copy(rows.at[t], acc.at[idxb.at[t * 2 + pb, 1]],
                       ssems[t], add=True)

    def wait_scatter(t, pb):
      pltpu.make_async_copy(rows.at[t], acc.at[idxb.at[t * 2 + pb, 1]],
                            ssems[t]).wait()

    for p in range(2):
      pltpu.sync_copy(x_hbm.at[p, rslice], x_sp.at[rslice])
      pltpu.sync_copy(zer_hbm.at[rslice], acc.at[rslice])
      plsc.subcore_barrier()

      for t in range(NSTREAM):
        pltpu.sync_copy(idx_hbm.at[c, s, t * CPS], idxb.at[t * 2])
        fire_gather(t, 0, p)
        fire_idx(t, 1, 1)

      @pl.loop(0, CPS // 2)
      def _(j):
        for par in range(2):
          for t in range(NSTREAM):
            wait_gather(t, par, p)
            fire_scatter(t, par)
          for t in range(NSTREAM):
            wait_scatter(t, par)
            if par == 0:
              wait_idx(t, 1)
              fire_gather(t, 1, p)
              @pl.when(j < CPS // 2 - 1)
              def _():
                fire_idx(t, 2 * j + 2, 0)
            else:
              @pl.when(j < CPS // 2 - 1)
              def _():
                wait_idx(t, 0)
                fire_gather(t, 0, p)
                fire_idx(t, 2 * j + 3, 1)

      plsc.subcore_barrier()
      pltpu.sync_copy(acc.at[rslice], out_hbm.at[c, p, rslice])
      plsc.subcore_barrier()

  return sc_kernel(x2, idx_t, zeros_init)


def _tc_body(a_ref, w1_ref, b1_ref, w2_ref, b2_ref, o_ref):
  a_lo = a_ref[0, 0, :N_NODES] + a_ref[1, 0, :N_NODES]
  a_hi = a_ref[0, 1, :N_NODES] + a_ref[1, 1, :N_NODES]
  w1_lo = w1_ref[:, :HALF]
  w1_hi = w1_ref[:, HALF:]
  h = (lax.dot_general(a_lo, w1_lo, (((1,), (1,)), ((), ())),
                       preferred_element_type=jnp.float32)
       + lax.dot_general(a_hi, w1_hi, (((1,), (1,)), ((), ())),
                         preferred_element_type=jnp.float32))
  h = jnp.maximum(h + b1_ref[...], 0.0)
  o = lax.dot_general(w2_ref[...], h, (((1,), (1,)), ((), ())),
                      preferred_element_type=jnp.float32)
  o_ref[...] = o + b2_ref[...]


def _tc_linear(agg4, W1, b1, W2, b2):
  return pl.pallas_call(
      _tc_body,
      out_shape=jax.ShapeDtypeStruct((DIM, N_NODES), jnp.float32),
  )(agg4, W1, b1.reshape(1, DIM), W2, b2.reshape(DIM, 1))


def kernel(x, edge_index, W1, b1, W2, b2):
  src = edge_index[0]
  dst = edge_index[1]
  pad = E_PAD - N_EDGES
  src_p = jnp.concatenate([src, jnp.zeros((pad,), jnp.int32)])
  dst_p = jnp.concatenate(
      [dst, jnp.full((pad,), N_NODES, jnp.int32)])
  src_t = src_p.reshape(NC, NS, CHUNKS_PER_TILE, 1, CHUNK)
  dst_t = dst_p.reshape(NC, NS, CHUNKS_PER_TILE, 1, CHUNK)
  idx_t = jnp.concatenate([src_t, dst_t], axis=3)
  x_pad = jnp.concatenate(
      [x, jnp.zeros((ACC_ROWS - N_NODES, DIM), jnp.float32)])
  x2 = x_pad.reshape(ACC_ROWS, 2, HALF).transpose(1, 0, 2)
  zeros_init = jnp.zeros((ACC_ROWS, HALF), jnp.float32)
  agg4 = _sc_aggregate(x2, idx_t, zeros_init)
  return _tc_linear(agg4, W1, b1, W2, b2)


# speedup vs baseline: 1.1167x; 1.1167x over previous
import functools

import jax
import jax.numpy as jnp
from jax import lax
from jax.experimental import pallas as pl
from jax.experimental.pallas import tpu as pltpu
from jax.experimental.pallas import tpu_sc as plsc

N_NODES = 10000
N_EDGES = 320000
DIM = 128
HALF = DIM // 2

NC = 2
NS = 16
CHUNK = 128
CHUNKS_PER_TILE = 80
EDGES_PER_TILE = CHUNK * CHUNKS_PER_TILE
E_PAD = NC * NS * EDGES_PER_TILE
ACC_ROWS = 10112
ROWS_PER_TILE = ACC_ROWS // NS

NSTREAM = 2
CPS = CHUNKS_PER_TILE // NSTREAM


def _sc_aggregate(x2, idx_t, zeros_init):
  mesh = plsc.VectorSubcoreMesh(
      core_axis_name="c", subcore_axis_name="s", num_cores=NC,
      num_subcores=NS)

  @functools.partial(
      pl.kernel,
      out_type=jax.ShapeDtypeStruct((NC, 2, ACC_ROWS, HALF), jnp.float32),
      mesh=mesh,
      scratch_types=[
          pltpu.VMEM_SHARED((ACC_ROWS, HALF), jnp.float32),
          pltpu.VMEM_SHARED((ACC_ROWS, HALF), jnp.float32),
          pltpu.VMEM((NSTREAM * 2, 2, CHUNK), jnp.int32),
          pltpu.VMEM((NSTREAM, CHUNK, HALF), jnp.float32),
          [pltpu.SemaphoreType.DMA] * NSTREAM,
          [pltpu.SemaphoreType.DMA] * NSTREAM,
          [pltpu.SemaphoreType.DMA] * NSTREAM,
      ],
      compiler_params=pltpu.CompilerParams(use_tc_tiling_on_sc=False),
  )
  def sc_kernel(x_hbm, idx_hbm, zer_hbm, out_hbm,
                x_sp, acc, idxb, rows, gsems, ssems, isems):
    c = lax.axis_index("c")
    s = lax.axis_index("s")
    rslice = pl.ds(s * ROWS_PER_TILE, ROWS_PER_TILE)

    def fire_idx(t, i, pb):
      pltpu.async_copy(idx_hbm.at[c, s, t * CPS + i],
                       idxb.at[t * 2 + pb], isems[t])

    def wait_idx(t, pb):
      pltpu.make_async_copy(idx_hbm.at[0, 0, 0], idxb.at[t * 2 + pb],
                            isems[t]).wait()

    def fire_gather(t, pb, p):
      pltpu.async_copy(x_sp.at[idxb.at[t * 2 + pb, 0]],
                       rows.at[t], gsems[t])

    def wait_gather(t, pb, p):
      pltpu.make_async_copy(x_sp.at[idxb.at[t * 2 + pb, 0]],
                            rows.at[t], gsems[t]).wait()

    def fire_scatter(t, pb):
      pltpu.async_copy(rows.at[t], acc.at[idxb.at[t * 2 + pb, 1]],
                       ssems[t], add=True)

    def wait_scatter(t, pb):
      pltpu.make_async_copy(rows.at[t], acc.at[idxb.at[t * 2 + pb, 1]],
                            ssems[t]).wait()

    for p in range(2):
      pltpu.sync_copy(x_hbm.at[p, rslice], x_sp.at[rslice])
      pltpu.sync_copy(zer_hbm.at[rslice], acc.at[rslice])
      plsc.subcore_barrier()

      for t in range(NSTREAM):
        pltpu.sync_copy(idx_hbm.at[c, s, t * CPS], idxb.at[t * 2])
        fire_gather(t, 0, p)
        fire_idx(t, 1, 1)

      @pl.loop(0, CPS // 2)
      def _(j):
        for par in range(2):
          for t in range(NSTREAM):
            wait_gather(t, par, p)
            fire_scatter(t, par)
          for t in range(NSTREAM):
            wait_scatter(t, par)
            if par == 0:
              wait_idx(t, 1)
              fire_gather(t, 1, p)
              @pl.when(j < CPS // 2 - 1)
              def _():
                fire_idx(t, 2 * j + 2, 0)
            else:
              @pl.when(j < CPS // 2 - 1)
              def _():
                wait_idx(t, 0)
                fire_gather(t, 0, p)
                fire_idx(t, 2 * j + 3, 1)

      plsc.subcore_barrier()
      pltpu.sync_copy(acc.at[rslice], out_hbm.at[c, p, rslice])
      plsc.subcore_barrier()

  return sc_kernel(x2, idx_t, zeros_init)


def _tc_body(a_ref, w1_ref, b1_ref, w2_ref, b2_ref, o_ref):
  a_lo = a_ref[0, 0, :N_NODES] + a_ref[1, 0, :N_NODES]
  a_hi = a_ref[0, 1, :N_NODES] + a_ref[1, 1, :N_NODES]
  w1_lo = w1_ref[:, :HALF]
  w1_hi = w1_ref[:, HALF:]
  h = (lax.dot_general(a_lo, w1_lo, (((1,), (1,)), ((), ())),
                       preferred_element_type=jnp.float32)
       + lax.dot_general(a_hi, w1_hi, (((1,), (1,)), ((), ())),
                         preferred_element_type=jnp.float32))
  h = jnp.maximum(h + b1_ref[...], 0.0)
  o = lax.dot_general(w2_ref[...], h, (((1,), (1,)), ((), ())),
                      preferred_element_type=jnp.float32)
  o_ref[...] = o + b2_ref[...]


def _tc_linear(agg4, W1, b1, W2, b2):
  return pl.pallas_call(
      _tc_body,
      out_shape=jax.ShapeDtypeStruct((DIM, N_NODES), jnp.float32),
  )(agg4, W1, b1.reshape(1, DIM), W2, b2.reshape(DIM, 1))


def kernel(x, edge_index, W1, b1, W2, b2):
  src = edge_index[0]
  dst = edge_index[1]
  pad = E_PAD - N_EDGES
  src_p = jnp.concatenate([src, jnp.zeros((pad,), jnp.int32)])
  dst_p = jnp.concatenate(
      [dst, jnp.full((pad,), N_NODES, jnp.int32)])
  src_t = src_p.reshape(NC, NS, CHUNKS_PER_TILE, 1, CHUNK)
  dst_t = dst_p.reshape(NC, NS, CHUNKS_PER_TILE, 1, CHUNK)
  idx_t = jnp.concatenate([src_t, dst_t], axis=3)
  x_pad = jnp.concatenate(
      [x, jnp.zeros((ACC_ROWS - N_NODES, DIM), jnp.float32)])
  x2 = x_pad.reshape(ACC_ROWS, 2, HALF).transpose(1, 0, 2)
  zeros_init = jnp.zeros((ACC_ROWS, HALF), jnp.float32)
  agg4 = _sc_aggregate(x2, idx_t, zeros_init)
  return _tc_linear(agg4, W1, b1, W2, b2)


# speedup vs baseline: 1.2294x; 1.1010x over previous
import functools

import jax
import jax.numpy as jnp
from jax import lax
from jax.experimental import pallas as pl
from jax.experimental.pallas import tpu as pltpu
from jax.experimental.pallas import tpu_sc as plsc

N_NODES = 10000
N_EDGES = 320000
DIM = 128
HALF = DIM // 2

NC = 2
NS = 16
CHUNK = 128
CHUNKS_PER_TILE = 80
EDGES_PER_TILE = CHUNK * CHUNKS_PER_TILE
E_PAD = NC * NS * EDGES_PER_TILE
ACC_ROWS = 10112
ROWS_PER_TILE = ACC_ROWS // NS
X_LAST_TILE = N_NODES - 15 * ROWS_PER_TILE

NSTREAM = 2
CPS = CHUNKS_PER_TILE // NSTREAM


def _sc_aggregate(x, src_t, dst_t, zeros_init):
  mesh = plsc.VectorSubcoreMesh(
      core_axis_name="c", subcore_axis_name="s", num_cores=NC,
      num_subcores=NS)

  @functools.partial(
      pl.kernel,
      out_type=jax.ShapeDtypeStruct((NC, 2, ACC_ROWS, HALF), jnp.float32),
      mesh=mesh,
      scratch_types=[
          pltpu.VMEM_SHARED((ACC_ROWS, HALF), jnp.float32),
          pltpu.VMEM_SHARED((ACC_ROWS, HALF), jnp.float32),
          pltpu.VMEM((CHUNKS_PER_TILE, CHUNK), jnp.int32),
          pltpu.VMEM((CHUNKS_PER_TILE, CHUNK), jnp.int32),
          pltpu.VMEM((NSTREAM, CHUNK, HALF), jnp.float32),
          [pltpu.SemaphoreType.DMA] * NSTREAM,
          [pltpu.SemaphoreType.DMA] * NSTREAM,
      ],
      compiler_params=pltpu.CompilerParams(use_tc_tiling_on_sc=False),
  )
  def sc_kernel(x_hbm, src_hbm, dst_hbm, zer_hbm, out_hbm,
                x_sp, acc, src_v, dst_v, rows, gsems, ssems):
    c = lax.axis_index("c")
    s = lax.axis_index("s")
    rslice = pl.ds(s * ROWS_PER_TILE, ROWS_PER_TILE)

    pltpu.sync_copy(src_hbm.at[c, s], src_v)
    pltpu.sync_copy(dst_hbm.at[c, s], dst_v)

    def fire_gather(t, i):
      pltpu.async_copy(x_sp.at[src_v.at[t * CPS + i]], rows.at[t],
                       gsems[t])

    def wait_gather(t, i):
      pltpu.make_async_copy(x_sp.at[src_v.at[t * CPS + i]],
                            rows.at[t], gsems[t]).wait()

    def fire_scatter(t, i):
      pltpu.async_copy(rows.at[t], acc.at[dst_v.at[t * CPS + i]],
                       ssems[t], add=True)

    def wait_scatter(t, i):
      pltpu.make_async_copy(rows.at[t], acc.at[dst_v.at[t * CPS + i]],
                            ssems[t]).wait()

    for p in range(2):
      cslice = pl.ds(p * HALF, HALF)
      @pl.when(s < NS - 1)
      def _():
        pltpu.sync_copy(
            x_hbm.at[pl.ds(s * ROWS_PER_TILE, ROWS_PER_TILE), cslice],
            x_sp.at[rslice])
      @pl.when(s == NS - 1)
      def _():
        pltpu.sync_copy(
            x_hbm.at[pl.ds(15 * ROWS_PER_TILE, X_LAST_TILE), cslice],
            x_sp.at[pl.ds(15 * ROWS_PER_TILE, X_LAST_TILE)])
      pltpu.sync_copy(zer_hbm.at[rslice], acc.at[rslice])
      plsc.subcore_barrier()

      for t in range(NSTREAM):
        fire_gather(t, 0)

      @pl.loop(0, CPS)
      def _(i):
        for t in range(NSTREAM):
          wait_gather(t, i)
          fire_scatter(t, i)
        for t in range(NSTREAM):
          wait_scatter(t, i)
          @pl.when(i < CPS - 1)
          def _():
            fire_gather(t, i + 1)

      plsc.subcore_barrier()
      pltpu.sync_copy(acc.at[rslice], out_hbm.at[c, p, rslice])
      plsc.subcore_barrier()

  return sc_kernel(x, src_t, dst_t, zeros_init)


def _tc_body(a_ref, w1_ref, b1_ref, w2_ref, b2_ref, o_ref):
  a_lo = a_ref[0, 0, :N_NODES] + a_ref[1, 0, :N_NODES]
  a_hi = a_ref[0, 1, :N_NODES] + a_ref[1, 1, :N_NODES]
  w1_lo = w1_ref[:, :HALF]
  w1_hi = w1_ref[:, HALF:]
  h = (lax.dot_general(a_lo, w1_lo, (((1,), (1,)), ((), ())),
                       preferred_element_type=jnp.float32)
       + lax.dot_general(a_hi, w1_hi, (((1,), (1,)), ((), ())),
                         preferred_element_type=jnp.float32))
  h = jnp.maximum(h + b1_ref[...], 0.0)
  o = lax.dot_general(w2_ref[...], h, (((1,), (1,)), ((), ())),
                      preferred_element_type=jnp.float32)
  o_ref[...] = o + b2_ref[...]


def _tc_linear(agg4, W1, b1, W2, b2):
  return pl.pallas_call(
      _tc_body,
      out_shape=jax.ShapeDtypeStruct((DIM, N_NODES), jnp.float32),
  )(agg4, W1, b1.reshape(1, DIM), W2, b2.reshape(DIM, 1))


def kernel(x, edge_index, W1, b1, W2, b2):
  src = edge_index[0]
  dst = edge_index[1]
  pad = E_PAD - N_EDGES
  src_t = jnp.concatenate(
      [src, jnp.zeros((pad,), jnp.int32)]).reshape(
          NC, NS, CHUNKS_PER_TILE, CHUNK)
  dst_t = jnp.concatenate(
      [dst, jnp.full((pad,), N_NODES, jnp.int32)]).reshape(
          NC, NS, CHUNKS_PER_TILE, CHUNK)
  zeros_init = jnp.zeros((ACC_ROWS, HALF), jnp.float32)
  agg4 = _sc_aggregate(x, src_t, dst_t, zeros_init)
  return _tc_linear(agg4, W1, b1, W2, b2)
